# baseline (speedup 1.0000x reference)
import functools

import jax
import jax.numpy as jnp
from jax import lax
from jax.experimental import pallas as pl
from jax.experimental.pallas import tpu as pltpu
from jax.experimental.pallas import tpu_sc as plsc

N_NODES = 10000
N_EDGES = 320000
D = 128

NC = 2
NS = 16
NW = NC * NS
CHUNK = 128
NROWS = N_EDGES // CHUNK
KB = 8
NBIG = 24
EPIW = 3
REMROWS = 4
REME = REMROWS * CHUNK
ROWS_PER_TILE = 640
TBL = NS * ROWS_PER_TILE
NDEG = 10240
L = 16
NBUF = 2


def _sc_aggregate(x, ei3, srcr, dstr, zeros_a, zeros_d):
    mesh = plsc.VectorSubcoreMesh(core_axis_name="c", subcore_axis_name="s")

    @functools.partial(
        pl.kernel,
        mesh=mesh,
        compiler_params=pltpu.CompilerParams(needs_layout_passes=False),
        out_type=[
            jax.ShapeDtypeStruct((NC * TBL, D), jnp.float32),
            jax.ShapeDtypeStruct((NW, NDEG), jnp.float32),
        ],
        scratch_types=[
            pltpu.VMEM((2, KB, CHUNK), jnp.int32),
            pltpu.VMEM((2, KB, CHUNK), jnp.int32),
            pltpu.VMEM((CHUNK,), jnp.int32),
            pltpu.VMEM((CHUNK,), jnp.int32),
            pltpu.VMEM((1, CHUNK), jnp.int32),
            pltpu.VMEM((NBUF, CHUNK, D), jnp.float32),
            pltpu.VMEM((NDEG,), jnp.float32),
            pltpu.SemaphoreType.DMA,
            pltpu.SemaphoreType.DMA,
            pltpu.VMEM_SHARED((TBL, D), jnp.float32),
        ],
    )
    def agg_kernel(x_hbm, ei_hbm, srcr_hbm, dstr_hbm, za_hbm, zd_hbm,
                   agg_out, deg_out,
                   src_v, dst_v, srce_v, dste_v, drow_v, rows_v, deg_v,
                   gsem, isem, agg_sh):
        c = lax.axis_index("c")
        s = lax.axis_index("s")
        wid = c * NS + s
        row0 = s * ROWS_PER_TILE
        small = lax.rem(wid, 4) == 3
        big = 1 - small.astype(jnp.int32)
        nsmall = wid // 4
        r0 = 80 * (wid - nsmall) + 72 * nsmall
        nblk = KB + 1 + big
        pltpu.sync_copy(za_hbm, agg_sh.at[pl.ds(row0, ROWS_PER_TILE)])
        pltpu.sync_copy(zd_hbm, deg_v)
        pltpu.sync_copy(ei_hbm.at[0, pl.ds(r0, KB)], src_v.at[0])
        pltpu.sync_copy(ei_hbm.at[1, pl.ds(r0, KB)], dst_v.at[0])
        plsc.subcore_barrier()

        ones16 = jnp.ones((L,), jnp.float32)

        def blk(g, carry):
            sl = lax.rem(g, 2)
            nsl = 1 - sl

            @pl.when(g + 1 < nblk)
            def _prefetch():
                nb = r0 + (g + 1) * KB
                pltpu.async_copy(ei_hbm.at[0, pl.ds(nb, KB)],
                                 src_v.at[nsl], isem)
                pltpu.async_copy(ei_hbm.at[1, pl.ds(nb, KB)],
                                 dst_v.at[nsl], isem)

            def body(j, carry2):
                @pl.when(j >= NBUF)
                def _consume():
                    i = j - NBUF
                    b = lax.rem(i, NBUF)
                    pltpu.make_async_copy(
                        x_hbm.at[pl.ds(0, CHUNK)], rows_v.at[b], gsem).wait()
                    for k in range(CHUNK // L):
                        dvec = dst_v[sl, i, pl.ds(k * L, L)]
                        plsc.addupdate_scatter(deg_v, [dvec], ones16)
                    pltpu.sync_copy(rows_v.at[b], agg_sh.at[dst_v.at[sl, i]],
                                    add=True)

                @pl.when(j < KB)
                def _fire():
                    b = lax.rem(j, NBUF)
                    pltpu.async_copy(x_hbm.at[src_v.at[sl, j]], rows_v.at[b],
                                     gsem)

                return carry2

            lax.fori_loop(0, KB + NBUF, body, 0)

            @pl.when(g + 1 < nblk)
            def _wait_prefetch():
                pltpu.make_async_copy(ei_hbm.at[0, pl.ds(0, KB)],
                                      src_v.at[nsl], isem).wait()
                pltpu.make_async_copy(ei_hbm.at[1, pl.ds(0, KB)],
                                      dst_v.at[nsl], isem).wait()

            return carry

        lax.fori_loop(0, nblk, blk, 0)

        @pl.when(wid == EPIW)
        def _epilogue():
            def rem_chunk(q, carry):
                off = q * CHUNK
                pltpu.sync_copy(srcr_hbm.at[pl.ds(off, CHUNK)], srce_v)
                pltpu.sync_copy(dstr_hbm.at[pl.ds(off, CHUNK)], dste_v)
                pltpu.async_copy(x_hbm.at[srce_v], rows_v.at[0], gsem)
                pltpu.make_async_copy(
                    x_hbm.at[pl.ds(0, CHUNK)], rows_v.at[0], gsem).wait()
                for k in range(CHUNK // L):
                    dvec = dste_v[pl.ds(k * L, L)]
                    plsc.addupdate_scatter(deg_v, [dvec], ones16)
                    drow_v[0, pl.ds(k * L, L)] = dvec
                pltpu.sync_copy(rows_v.at[0], agg_sh.at[drow_v.at[0]],
                                add=True)
                return carry

            lax.fori_loop(0, REMROWS, rem_chunk, 0)

        plsc.subcore_barrier()
        out0 = c * TBL + row0
        pltpu.sync_copy(agg_sh.at[pl.ds(row0, ROWS_PER_TILE)],
                        agg_out.at[pl.ds(out0, ROWS_PER_TILE)])
        pltpu.sync_copy(deg_v, deg_out.at[wid])

    return agg_kernel(x, ei3, srcr, dstr, zeros_a, zeros_d)


BM = 1024


def _tc_root_body(xb, rw, b, out):
    out[...] = jnp.dot(xb[...], rw[...],
                       preferred_element_type=jnp.float32) + b[...]


def _tc_root(x, rw, b):
    return pl.pallas_call(
        _tc_root_body,
        grid=(pl.cdiv(N_NODES, BM),),
        in_specs=[
            pl.BlockSpec((BM, D), lambda i: (i, 0)),
            pl.BlockSpec((D, D), lambda i: (0, 0)),
            pl.BlockSpec((1, D), lambda i: (0, 0)),
        ],
        out_specs=pl.BlockSpec((BM, D), lambda i: (i, 0)),
        out_shape=jax.ShapeDtypeStruct((N_NODES, D), jnp.float32),
    )(x, rw, b)


def _tc_body(p0, p1, dg, xrw, w, out):
    deg = jnp.sum(dg[...], axis=0)[:, None]
    inv = 1.0 / jnp.maximum(deg, 1.0)
    mean = (p0[...] + p1[...]) * inv
    out[...] = jnp.dot(mean, w[...],
                       preferred_element_type=jnp.float32) + xrw[...]


def _tc_combine(agg, dg, xrw, w):
    nb = TBL // BM
    return pl.pallas_call(
        _tc_body,
        grid=(pl.cdiv(N_NODES, BM),),
        in_specs=[
            pl.BlockSpec((BM, D), lambda i: (i, 0)),
            pl.BlockSpec((BM, D), lambda i: (nb + i, 0)),
            pl.BlockSpec((NW, BM), lambda i: (0, i)),
            pl.BlockSpec((BM, D), lambda i: (i, 0)),
            pl.BlockSpec((D, D), lambda i: (0, 0)),
        ],
        out_specs=pl.BlockSpec((BM, D), lambda i: (i, 0)),
        out_shape=jax.ShapeDtypeStruct((N_NODES, D), jnp.float32),
    )(agg, agg, dg, xrw, w)


def kernel(x, edge_index, weight, root_weight, bias):
    ei = edge_index.astype(jnp.int32)
    ei3 = ei.reshape(2, NROWS, CHUNK)
    srcr = ei[0, N_EDGES - REME:]
    dstr = ei[1, N_EDGES - REME:]
    zeros_a = jnp.zeros((ROWS_PER_TILE, D), jnp.float32)
    zeros_d = jnp.zeros((NDEG,), jnp.float32)
    b2 = bias.reshape(1, D).astype(jnp.float32)
    xrw = _tc_root(x, root_weight, b2)
    agg, deg = _sc_aggregate(x, ei3, srcr, dstr, zeros_a, zeros_d)
    return _tc_combine(agg, deg, xrw, weight)


# speedup vs baseline: 1.0484x; 1.0484x over previous
import functools

import jax
import jax.numpy as jnp
from jax import lax
from jax.experimental import pallas as pl
from jax.experimental.pallas import tpu as pltpu
from jax.experimental.pallas import tpu_sc as plsc

N_NODES = 10000
N_EDGES = 320000
D = 128

NC = 2
NS = 16
NW = NC * NS
CHUNK = 128
NROWS = N_EDGES // CHUNK
KB = 8
NBIG = 24
EPIW = 3
REMROWS = 4
REME = REMROWS * CHUNK
ROWS_PER_TILE = 640
TBL = NS * ROWS_PER_TILE
NDEG = 10240
L = 16
NBUF = 2


def _sc_aggregate(x, ei3, srcr, dstr):
    mesh = plsc.VectorSubcoreMesh(core_axis_name="c", subcore_axis_name="s")

    @functools.partial(
        pl.kernel,
        mesh=mesh,
        compiler_params=pltpu.CompilerParams(needs_layout_passes=False),
        out_type=[
            jax.ShapeDtypeStruct((NC * TBL, D), jnp.float32),
            jax.ShapeDtypeStruct((NW, NDEG), jnp.float32),
        ],
        scratch_types=[
            pltpu.VMEM((2, KB, CHUNK), jnp.int32),
            pltpu.VMEM((2, KB, CHUNK), jnp.int32),
            pltpu.VMEM((CHUNK,), jnp.int32),
            pltpu.VMEM((CHUNK,), jnp.int32),
            pltpu.VMEM((1, CHUNK), jnp.int32),
            pltpu.VMEM((NBUF, CHUNK, D), jnp.float32),
            pltpu.VMEM((NDEG,), jnp.float32),
            pltpu.SemaphoreType.DMA,
            pltpu.SemaphoreType.DMA,
            pltpu.VMEM_SHARED((TBL, D), jnp.float32),
        ],
    )
    def agg_kernel(x_hbm, ei_hbm, srcr_hbm, dstr_hbm,
                   agg_out, deg_out,
                   src_v, dst_v, srce_v, dste_v, drow_v, rows_v, deg_v,
                   gsem, isem, agg_sh):
        c = lax.axis_index("c")
        s = lax.axis_index("s")
        wid = c * NS + s
        row0 = s * ROWS_PER_TILE
        small = lax.rem(wid, 4) == 3
        big = 1 - small.astype(jnp.int32)
        nsmall = wid // 4
        r0 = 80 * (wid - nsmall) + 72 * nsmall
        nblk = KB + 1 + big
        ones16 = jnp.ones((L,), jnp.float32)
        zero16 = jnp.zeros((L,), jnp.float32)

        def zrow(r, carry):
            for k in range(D // L):
                rows_v[0, r, pl.ds(k * L, L)] = zero16
            return carry

        lax.fori_loop(0, CHUNK, zrow, 0)
        for q in range(ROWS_PER_TILE // CHUNK):
            pltpu.async_copy(
                rows_v.at[0],
                agg_sh.at[pl.ds(row0 + q * CHUNK, CHUNK)], isem)

        def zdeg(i, carry):
            deg_v[pl.ds(i * L, L)] = zero16
            return carry

        lax.fori_loop(0, NDEG // L, zdeg, 0)
        pltpu.sync_copy(ei_hbm.at[0, pl.ds(r0, KB)], src_v.at[0])
        pltpu.sync_copy(ei_hbm.at[1, pl.ds(r0, KB)], dst_v.at[0])
        for q in range(ROWS_PER_TILE // CHUNK):
            pltpu.make_async_copy(
                rows_v.at[0],
                agg_sh.at[pl.ds(row0 + q * CHUNK, CHUNK)], isem).wait()
        plsc.subcore_barrier()

        def blk(g, carry):
            sl = lax.rem(g, 2)
            nsl = 1 - sl

            @pl.when(g + 1 < nblk)
            def _prefetch():
                nb = r0 + (g + 1) * KB
                pltpu.async_copy(ei_hbm.at[0, pl.ds(nb, KB)],
                                 src_v.at[nsl], isem)
                pltpu.async_copy(ei_hbm.at[1, pl.ds(nb, KB)],
                                 dst_v.at[nsl], isem)

            def body(j, carry2):
                @pl.when(j >= NBUF)
                def _consume():
                    i = j - NBUF
                    b = lax.rem(i, NBUF)
                    pltpu.make_async_copy(
                        x_hbm.at[pl.ds(0, CHUNK)], rows_v.at[b], gsem).wait()
                    for k in range(CHUNK // L):
                        dvec = dst_v[sl, i, pl.ds(k * L, L)]
                        plsc.addupdate_scatter(deg_v, [dvec], ones16)
                    pltpu.sync_copy(rows_v.at[b], agg_sh.at[dst_v.at[sl, i]],
                                    add=True)

                @pl.when(j < KB)
                def _fire():
                    b = lax.rem(j, NBUF)
                    pltpu.async_copy(x_hbm.at[src_v.at[sl, j]], rows_v.at[b],
                                     gsem)

                return carry2

            lax.fori_loop(0, KB + NBUF, body, 0)

            @pl.when(g + 1 < nblk)
            def _wait_prefetch():
                pltpu.make_async_copy(ei_hbm.at[0, pl.ds(0, KB)],
                                      src_v.at[nsl], isem).wait()
                pltpu.make_async_copy(ei_hbm.at[1, pl.ds(0, KB)],
                                      dst_v.at[nsl], isem).wait()

            return carry

        lax.fori_loop(0, nblk, blk, 0)

        @pl.when(wid == EPIW)
        def _epilogue():
            def rem_chunk(q, carry):
                off = q * CHUNK
                pltpu.sync_copy(srcr_hbm.at[pl.ds(off, CHUNK)], srce_v)
                pltpu.sync_copy(dstr_hbm.at[pl.ds(off, CHUNK)], dste_v)
                pltpu.async_copy(x_hbm.at[srce_v], rows_v.at[0], gsem)
                pltpu.make_async_copy(
                    x_hbm.at[pl.ds(0, CHUNK)], rows_v.at[0], gsem).wait()
                for k in range(CHUNK // L):
                    dvec = dste_v[pl.ds(k * L, L)]
                    plsc.addupdate_scatter(deg_v, [dvec], ones16)
                    drow_v[0, pl.ds(k * L, L)] = dvec
                pltpu.sync_copy(rows_v.at[0], agg_sh.at[drow_v.at[0]],
                                add=True)
                return carry

            lax.fori_loop(0, REMROWS, rem_chunk, 0)

        plsc.subcore_barrier()
        out0 = c * TBL + row0
        pltpu.sync_copy(agg_sh.at[pl.ds(row0, ROWS_PER_TILE)],
                        agg_out.at[pl.ds(out0, ROWS_PER_TILE)])
        pltpu.sync_copy(deg_v, deg_out.at[wid])

    return agg_kernel(x, ei3, srcr, dstr)


BM = 1024


def _tc_body(p0, p1, dg, xb, w, rw, b, out):
    deg = jnp.sum(dg[...], axis=0)[:, None]
    inv = 1.0 / jnp.maximum(deg, 1.0)
    mean = (p0[...] + p1[...]) * inv
    out[...] = (jnp.dot(mean, w[...], preferred_element_type=jnp.float32)
                + jnp.dot(xb[...], rw[...], preferred_element_type=jnp.float32)
                + b[...])


def _tc_combine(agg, dg, x, w, rw, b):
    nb = TBL // BM
    return pl.pallas_call(
        _tc_body,
        grid=(pl.cdiv(N_NODES, BM),),
        in_specs=[
            pl.BlockSpec((BM, D), lambda i: (i, 0)),
            pl.BlockSpec((BM, D), lambda i: (nb + i, 0)),
            pl.BlockSpec((NW, BM), lambda i: (0, i)),
            pl.BlockSpec((BM, D), lambda i: (i, 0)),
            pl.BlockSpec((D, D), lambda i: (0, 0)),
            pl.BlockSpec((D, D), lambda i: (0, 0)),
            pl.BlockSpec((1, D), lambda i: (0, 0)),
        ],
        out_specs=pl.BlockSpec((BM, D), lambda i: (i, 0)),
        out_shape=jax.ShapeDtypeStruct((N_NODES, D), jnp.float32),
    )(agg, agg, dg, x, w, rw, b)


def kernel(x, edge_index, weight, root_weight, bias):
    ei = edge_index.astype(jnp.int32)
    ei3 = ei.reshape(2, NROWS, CHUNK)
    srcr = ei[0, N_EDGES - REME:]
    dstr = ei[1, N_EDGES - REME:]
    agg, deg = _sc_aggregate(x, ei3, srcr, dstr)
    b2 = bias.reshape(1, D).astype(jnp.float32)
    return _tc_combine(agg, deg, x, weight, root_weight, b2)
